# Initial kernel scaffold; baseline (speedup 1.0000x reference)
import jax
import jax.numpy as jnp
from jax.experimental import pallas as pl


def kernel(positions, cell, mask_triples, offsets, etas, neighbors_j, neighbors_k, offsets_j, offsets_k, atomic_numbers):
    raise NotImplementedError("write your pallas kernel here")


# speedup vs baseline: 182.3758x; 182.3758x over previous
import functools

import jax
import jax.numpy as jnp
from jax import lax
from jax.experimental import pallas as pl
from jax.experimental.pallas import tpu as pltpu
from jax.experimental.pallas import tpu_sc as plsc

_CUT_COEF = (1.0, -4.9348011, 4.0586948, -1.3351585,
             0.23502980, -0.025358984, 0.0015939107)


def _rsqrt(x):
    i = plsc.bitcast(x, jnp.int32)
    y = plsc.bitcast(jnp.int32(0x5F3759DF) - (i >> 1), jnp.float32)
    for _ in range(3):
        y = y * (1.5 - 0.5 * x * y * y)
    return y


def _cutoff_sq(s, cut2):
    u = s * (1.0 / cut2)
    acc = jnp.full_like(u, _CUT_COEF[-1])
    for c in _CUT_COEF[-2::-1]:
        acc = acc * u + c
    val = 0.5 * acc + 0.5
    return jnp.where(s < cut2, val, 0.0)


def _build(B, A, T, NN, E):
    info = plsc.get_sparse_core_info()
    NC, NS, L = info.num_cores, info.num_subcores, info.num_lanes
    NW = NC * NS
    ROWS = B * A
    RPW = ROWS // NW
    GROUPS = RPW // L
    CUT2 = 25.0

    mesh = plsc.VectorSubcoreMesh(core_axis_name="c", subcore_axis_name="s")

    @functools.partial(
        pl.kernel, mesh=mesh,
        out_type=jax.ShapeDtypeStruct((B, A * 2 * E), jnp.float32),
        compiler_params=pltpu.CompilerParams(needs_layout_passes=False),
        scratch_types=[
            pltpu.VMEM((A * 3,), jnp.float32),
            pltpu.VMEM((A,), jnp.int32),
            pltpu.VMEM((128,), jnp.float32),
            pltpu.VMEM((L * T,), jnp.int32),
            pltpu.VMEM((L * T,), jnp.int32),
            pltpu.VMEM((L * T,), jnp.int32),
            pltpu.VMEM((L * T,), jnp.int32),
            pltpu.VMEM((L * T,), jnp.float32),
            pltpu.VMEM((L * NN * 3,), jnp.float32),
            pltpu.VMEM((3 * L * NN,), jnp.float32),
            pltpu.VMEM((RPW * 2 * E,), jnp.float32),
        ],
    )
    def behler_g2(pos_h, aux_h, mask_h, offs_h, nj_h, nk_h, oj_h,
                  ok_h, z_h, out_h, pos_v, z_v, aux_v, nj_v, nk_v,
                  oj_v, ok_v, mask_v, offr_v, offc_v, out_v):
        wid = lax.axis_index("s") * NC + lax.axis_index("c")
        row0 = wid * RPW
        b = row0 // A
        a0 = row0 - b * A

        pltpu.sync_copy(pos_h.at[b], pos_v)
        pltpu.sync_copy(z_h.at[b], z_v)
        pltpu.sync_copy(aux_h.at[b], aux_v)

        lane = jnp.arange(L, dtype=jnp.int32)
        zero = jnp.zeros((L,), jnp.int32)

        zf = jnp.zeros((L,), jnp.float32)
        v0 = aux_v[pl.ds(0, L)]
        cell_s = [[zf + v0[3 * d + c] for c in range(3)] for d in range(3)]
        neta = [zf - v0[9 + e] for e in range(E)]

        def group_body(g, _):
            ab = a0 + g * L
            pltpu.sync_copy(nj_h.at[b, pl.ds(ab * T, L * T)], nj_v)
            pltpu.sync_copy(nk_h.at[b, pl.ds(ab * T, L * T)], nk_v)
            pltpu.sync_copy(oj_h.at[b, pl.ds(ab * T, L * T)], oj_v)
            pltpu.sync_copy(ok_h.at[b, pl.ds(ab * T, L * T)], ok_v)
            pltpu.sync_copy(mask_h.at[b, pl.ds(ab * T, L * T)], mask_v)
            pltpu.sync_copy(offs_h.at[b, pl.ds(ab * NN * 3, L * NN * 3)],
                            offr_v)

            def off_body(p, _):
                pv3 = (p * L + lane) * 3
                ox = plsc.load_gather(offr_v, [pv3])
                oy = plsc.load_gather(offr_v, [pv3 + 1])
                oz = plsc.load_gather(offr_v, [pv3 + 2])
                for c in range(3):
                    val = (ox * cell_s[0][c] + oy * cell_s[1][c]
                           + oz * cell_s[2][c])
                    offc_v[pl.ds(c * L * NN + p * L, L)] = val
                return 0

            lax.fori_loop(0, (L * NN) // L, off_body, 0)

            civ3 = (ab + lane) * 3
            pix = plsc.load_gather(pos_v, [civ3])
            piy = plsc.load_gather(pos_v, [civ3 + 1])
            piz = plsc.load_gather(pos_v, [civ3 + 2])

            def t_body(t, accs):
                lt = lane * T + t
                njv = plsc.load_gather(nj_v, [lt])
                nkv = plsc.load_gather(nk_v, [lt])
                ojv = plsc.load_gather(oj_v, [lt])
                okv = plsc.load_gather(ok_v, [lt])
                mkv = plsc.load_gather(mask_v, [lt])

                nj3 = njv * 3
                nk3 = nkv * 3
                pjx = plsc.load_gather(pos_v, [nj3])
                pjy = plsc.load_gather(pos_v, [nj3 + 1])
                pjz = plsc.load_gather(pos_v, [nj3 + 2])
                pkx = plsc.load_gather(pos_v, [nk3])
                pky = plsc.load_gather(pos_v, [nk3 + 1])
                pkz = plsc.load_gather(pos_v, [nk3 + 2])
                zj = plsc.load_gather(z_v, [njv]).astype(jnp.float32)
                zk = plsc.load_gather(z_v, [nkv]).astype(jnp.float32)

                fj = lane * NN + ojv
                fk = lane * NN + okv
                pjx = pjx + plsc.load_gather(offc_v, [fj])
                pjy = pjy + plsc.load_gather(offc_v, [fj + L * NN])
                pjz = pjz + plsc.load_gather(offc_v, [fj + 2 * L * NN])
                pkx = pkx + plsc.load_gather(offc_v, [fk])
                pky = pky + plsc.load_gather(offc_v, [fk + L * NN])
                pkz = pkz + plsc.load_gather(offc_v, [fk + 2 * L * NN])

                djx, djy, djz = pjx - pix, pjy - piy, pjz - piz
                dkx, dky, dkz = pkx - pix, pky - piy, pkz - piz
                dx, dy, dz = pjx - pkx, pjy - pky, pjz - pkz
                sij = djx * djx + djy * djy + djz * djz + 1e-12
                sik = dkx * dkx + dky * dky + dkz * dkz + 1e-12
                sjk = dx * dx + dy * dy + dz * dz + 1e-12

                ssum = sij + sik
                S = ssum + sjk
                cost = (ssum - sjk) * 0.5 * _rsqrt(sij * sik)
                w = ((1.0 - cost) * zj * zk
                     * _cutoff_sq(sij, CUT2) * _cutoff_sq(sik, CUT2)
                     * _cutoff_sq(sjk, CUT2))
                w = jnp.where(mkv == 0.0, 0.0, w)
                return tuple(accs[e] + w * jnp.exp(neta[e] * S)
                             for e in range(E))

            accs = lax.fori_loop(0, T, t_body, (zf,) * E)

            r2e = (g * L + lane) * (2 * E)
            for e in range(E):
                plsc.store_scatter(out_v, [r2e + (2 * e)], accs[e])
                plsc.store_scatter(out_v, [r2e + (2 * e + 1)], 4.0 * accs[e])
            return 0

        lax.fori_loop(0, GROUPS, group_body, 0)
        pltpu.sync_copy(out_v, out_h.at[b, pl.ds(a0 * 2 * E, RPW * 2 * E)])

    return behler_g2


def kernel(positions, cell, mask_triples, offsets, etas, neighbors_j,
           neighbors_k, offsets_j, offsets_k, atomic_numbers):
    B, A, T = neighbors_j.shape
    NN = offsets.shape[2]
    E = etas.shape[0]
    fn = _build(B, A, T, NN, E)
    aux = jnp.concatenate(
        [cell.reshape(B, 9), jnp.broadcast_to(etas, (B, E)),
         jnp.zeros((B, 128 - 9 - E), jnp.float32)], axis=1)
    out = fn(positions.reshape(B, A * 3), aux,
             mask_triples.reshape(B, A * T), offsets.reshape(B, A * NN * 3),
             neighbors_j.astype(jnp.int32).reshape(B, A * T),
             neighbors_k.astype(jnp.int32).reshape(B, A * T),
             offsets_j.astype(jnp.int32).reshape(B, A * T),
             offsets_k.astype(jnp.int32).reshape(B, A * T),
             atomic_numbers.astype(jnp.int32))
    return out.reshape(B, A, 2 * E)
